# baseline (speedup 1.0000x reference)
import functools

import jax
import jax.numpy as jnp
from jax import lax
from jax.experimental import pallas as pl
from jax.experimental.pallas import tpu as pltpu
from jax.experimental.pallas import tpu_sc as plsc

_SEQ = 2048
_NTOK = _SEQ * _SEQ
_D = 64
_DP = 128
_NC = 2
_NS = 16
_NW = _NC * _NS
_PER_W = _NTOK // _NW
_ILIM = 128
_GROUP = 128
_NBUF = 4
_SUP_ROWS = 16
_SUP = _SUP_ROWS * _ILIM
_GPS = _SUP // _GROUP
_NSUP = _PER_W // _SUP


def kernel(relative_position, emb):
    idx2 = relative_position.reshape(_NTOK // _ILIM, _ILIM)
    emb_p = jnp.pad(emb, ((0, 0), (0, _DP - _D)))
    mesh = plsc.VectorSubcoreMesh(core_axis_name="c", subcore_axis_name="s")

    @functools.partial(
        pl.kernel,
        out_type=jax.ShapeDtypeStruct((_SEQ, _SEQ, _DP), jnp.float32),
        mesh=mesh,
        compiler_params=pltpu.CompilerParams(use_tc_tiling_on_sc=True),
        scratch_types=[
            pltpu.VMEM((_SUP_ROWS, _ILIM), jnp.int32),
            pltpu.VMEM((_SUP_ROWS, _ILIM), jnp.int32),
            pltpu.VMEM((_GROUP, _DP), jnp.float32),
            pltpu.VMEM((_GROUP, _DP), jnp.float32),
            pltpu.VMEM((_GROUP, _DP), jnp.float32),
            pltpu.VMEM((_GROUP, _DP), jnp.float32),
            pltpu.VMEM_SHARED((257, _DP), jnp.float32),
            pltpu.SemaphoreType.DMA,
            pltpu.SemaphoreType.DMA,
            pltpu.SemaphoreType.DMA,
            pltpu.SemaphoreType.DMA,
            pltpu.SemaphoreType.DMA,
            pltpu.SemaphoreType.DMA,
            pltpu.SemaphoreType.DMA,
            pltpu.SemaphoreType.DMA,
            pltpu.SemaphoreType.DMA,
            pltpu.SemaphoreType.DMA,
        ],
    )
    def _lookup(idx_hbm, emb_hbm, out_hbm, ibuf0, ibuf1, rows0, rows1,
                rows2, rows3, tbl_s, si0, si1, sg0, sg1, sg2, sg3,
                sw0, sw1, sw2, sw3):
        wid = lax.axis_index("s") * _NC + lax.axis_index("c")
        base = wid * _PER_W
        ibase = pl.multiple_of(base // _ILIM, _SUP_ROWS)
        si = (si0, si1)
        sg = (sg0, sg1, sg2, sg3)
        sw = (sw0, sw1, sw2, sw3)
        ibuf = (ibuf0, ibuf1)
        rows = (rows0, rows1, rows2, rows3)

        @pl.when(lax.axis_index("s") == 0)
        def _():
            pltpu.sync_copy(emb_hbm, tbl_s)

        plsc.subcore_barrier()

        def idx_load(sup, ss):
            start = pl.multiple_of(ibase + sup * _SUP_ROWS, _SUP_ROWS)
            pltpu.async_copy(
                idx_hbm.at[pl.ds(start, _SUP_ROWS)], ibuf[ss], si[ss])

        idx_load(0, 0)
        idx_load(1, 1)

        def sup_body(s2, carry):
            for ss in (0, 1):
                s = s2 * 2 + ss
                pltpu.make_async_copy(
                    idx_hbm.at[pl.ds(ibase, _SUP_ROWS)], ibuf[ss],
                    si[ss]).wait()

                def grp_body(q4, carry2):
                    for b in range(_NBUF):
                        q = q4 * _NBUF + b
                        g = s * _GPS + q
                        off = base + g * _GROUP
                        row_i = off // _SEQ
                        col_j = pl.multiple_of(off % _SEQ, _GROUP)

                        @pl.when(g >= _NBUF)
                        def _():
                            pltpu.make_async_copy(
                                rows[b],
                                out_hbm.at[0, pl.ds(0, _GROUP)],
                                sw[b]).wait()

                        pltpu.async_copy(
                            tbl_s.at[ibuf[ss].at[q]], rows[b], sg[b])
                        pltpu.make_async_copy(
                            tbl_s.at[ibuf[ss].at[q]], rows[b],
                            sg[b]).wait()
                        pltpu.async_copy(
                            rows[b],
                            out_hbm.at[row_i, pl.ds(col_j, _GROUP)],
                            sw[b])
                    return carry2

                lax.fori_loop(0, _GPS // _NBUF, grp_body, 0)

                idx_load(jnp.minimum(s + 2, _NSUP - 1), ss)
            return carry

        lax.fori_loop(0, _NSUP // 2, sup_body, 0)

        for ss in (0, 1):
            pltpu.make_async_copy(
                idx_hbm.at[pl.ds(ibase, _SUP_ROWS)], ibuf[ss], si[ss]).wait()
        for b in range(_NBUF):
            pltpu.make_async_copy(
                rows[b], out_hbm.at[0, pl.ds(0, _GROUP)], sw[b]).wait()

    out = _lookup(idx2, emb_p)
    return out[:, :, :_D]


# speedup vs baseline: 1.0260x; 1.0260x over previous
import functools

import jax
import jax.numpy as jnp
from jax import lax
from jax.experimental import pallas as pl
from jax.experimental.pallas import tpu as pltpu
from jax.experimental.pallas import tpu_sc as plsc

_SEQ = 2048
_NTOK = _SEQ * _SEQ
_D = 64
_DP = 128
_NC = 2
_NS = 16
_NW = _NC * _NS
_PER_W = _NTOK // _NW
_ILIM = 128
_GROUP = 256
_GPG = _GROUP // _ILIM
_SUP_ROWS = 16
_SUP = _SUP_ROWS * _ILIM
_GPS = _SUP // _GROUP
_NSUP = _PER_W // _SUP


def kernel(relative_position, emb):
    idx2 = relative_position.reshape(_NTOK // _ILIM, _ILIM)
    emb_p = jnp.pad(emb, ((0, 0), (0, _DP - _D)))
    mesh = plsc.VectorSubcoreMesh(core_axis_name="c", subcore_axis_name="s")

    @functools.partial(
        pl.kernel,
        out_type=jax.ShapeDtypeStruct((_SEQ, _SEQ, _DP), jnp.float32),
        mesh=mesh,
        compiler_params=pltpu.CompilerParams(use_tc_tiling_on_sc=True),
        scratch_types=[
            pltpu.VMEM((_SUP_ROWS, _ILIM), jnp.int32),
            pltpu.VMEM((_SUP_ROWS, _ILIM), jnp.int32),
            pltpu.VMEM((_GROUP, _DP), jnp.float32),
            pltpu.VMEM((_GROUP, _DP), jnp.float32),
            pltpu.VMEM_SHARED((257, _DP), jnp.float32),
            pltpu.SemaphoreType.DMA,
            pltpu.SemaphoreType.DMA,
            pltpu.SemaphoreType.DMA,
            pltpu.SemaphoreType.DMA,
            pltpu.SemaphoreType.DMA,
            pltpu.SemaphoreType.DMA,
        ],
    )
    def _lookup(idx_hbm, emb_hbm, out_hbm, ibuf0, ibuf1, rows0, rows1,
                tbl_s, si0, si1, sg0, sg1, sw0, sw1):
        wid = lax.axis_index("s") * _NC + lax.axis_index("c")
        base = wid * _PER_W
        ibase = pl.multiple_of(base // _ILIM, _SUP_ROWS)
        si = (si0, si1)
        sg = (sg0, sg1)
        sw = (sw0, sw1)
        ibuf = (ibuf0, ibuf1)
        rows = (rows0, rows1)

        @pl.when(lax.axis_index("s") == 0)
        def _():
            pltpu.sync_copy(emb_hbm, tbl_s)

        plsc.subcore_barrier()

        def idx_load(sup, ss):
            start = pl.multiple_of(ibase + sup * _SUP_ROWS, _SUP_ROWS)
            pltpu.async_copy(
                idx_hbm.at[pl.ds(start, _SUP_ROWS)], ibuf[ss], si[ss])

        idx_load(0, 0)
        idx_load(1, 1)

        def sup_body(s2, carry):
            for ss in (0, 1):
                s = s2 * 2 + ss
                pltpu.make_async_copy(
                    idx_hbm.at[pl.ds(ibase, _SUP_ROWS)], ibuf[ss],
                    si[ss]).wait()

                def grp_body(q2, carry2):
                    for b in (0, 1):
                        q = q2 * 2 + b
                        g = s * _GPS + q
                        off = base + g * _GROUP
                        row_i = off // _SEQ
                        col_j = pl.multiple_of(off % _SEQ, _GROUP)
                        qs = q * _GPG

                        @pl.when(g >= 2)
                        def _():
                            pltpu.make_async_copy(
                                rows[b],
                                out_hbm.at[0, pl.ds(0, _GROUP)],
                                sw[b]).wait()

                        for j in range(_GPG):
                            pltpu.async_copy(
                                tbl_s.at[ibuf[ss].at[qs + j]],
                                rows[b].at[pl.ds(j * _ILIM, _ILIM)], sg[b])
                        for j in range(_GPG):
                            pltpu.make_async_copy(
                                tbl_s.at[ibuf[ss].at[qs + j]],
                                rows[b].at[pl.ds(j * _ILIM, _ILIM)],
                                sg[b]).wait()
                        h = _GROUP // 2
                        pltpu.async_copy(
                            rows[b].at[pl.ds(0, h)],
                            out_hbm.at[row_i, pl.ds(col_j, h)],
                            sw[b])
                        pltpu.async_copy(
                            rows[b].at[pl.ds(h, h)],
                            out_hbm.at[row_i, pl.ds(col_j + h, h)],
                            sw[b])
                    return carry2

                lax.fori_loop(0, _GPS // 2, grp_body, 0)

                idx_load(jnp.minimum(s + 2, _NSUP - 1), ss)
            return carry

        lax.fori_loop(0, _NSUP // 2, sup_body, 0)

        for b in (0, 1):
            pltpu.make_async_copy(
                idx_hbm.at[pl.ds(ibase, _SUP_ROWS)], ibuf[b], si[b]).wait()
            pltpu.make_async_copy(
                rows[b], out_hbm.at[0, pl.ds(0, _GROUP)], sw[b]).wait()

    out = _lookup(idx2, emb_p)
    return out[:, :, :_D]


# speedup vs baseline: 1.0264x; 1.0004x over previous
import functools

import jax
import jax.numpy as jnp
from jax import lax
from jax.experimental import pallas as pl
from jax.experimental.pallas import tpu as pltpu
from jax.experimental.pallas import tpu_sc as plsc

_SEQ = 2048
_NTOK = _SEQ * _SEQ
_D = 64
_DP = 128
_NC = 2
_NS = 16
_NW = _NC * _NS
_PER_W = _NTOK // _NW
_ILIM = 128
_GROUP = 256
_GPG = _GROUP // _ILIM
_SUP_ROWS = 16
_SUP = _SUP_ROWS * _ILIM
_GPS = _SUP // _GROUP
_NSUP = _PER_W // _SUP


def kernel(relative_position, emb):
    idx2 = relative_position.reshape(_NTOK // _ILIM, _ILIM)
    emb_p = jnp.pad(emb, ((0, 0), (0, _DP - _D)))
    mesh = plsc.VectorSubcoreMesh(core_axis_name="c", subcore_axis_name="s")

    @functools.partial(
        pl.kernel,
        out_type=jax.ShapeDtypeStruct((_SEQ, _SEQ, _DP), jnp.float32),
        mesh=mesh,
        compiler_params=pltpu.CompilerParams(use_tc_tiling_on_sc=True),
        scratch_types=[
            pltpu.VMEM((_SUP_ROWS, _ILIM), jnp.int32),
            pltpu.VMEM((_SUP_ROWS, _ILIM), jnp.int32),
            pltpu.VMEM((_GROUP, _DP), jnp.float32),
            pltpu.VMEM((_GROUP, _DP), jnp.float32),
            pltpu.VMEM_SHARED((257, _DP), jnp.float32),
            pltpu.SemaphoreType.DMA,
            pltpu.SemaphoreType.DMA,
            pltpu.SemaphoreType.DMA,
            pltpu.SemaphoreType.DMA,
            pltpu.SemaphoreType.DMA,
            pltpu.SemaphoreType.DMA,
        ],
    )
    def _lookup(idx_hbm, emb_hbm, out_hbm, ibuf0, ibuf1, rows0, rows1,
                tbl_s, si0, si1, sg0, sg1, sw0, sw1):
        wid = lax.axis_index("s") * _NC + lax.axis_index("c")
        base = wid * _PER_W
        ibase = pl.multiple_of(base // _ILIM, _SUP_ROWS)
        si = (si0, si1)
        sg = (sg0, sg1)
        sw = (sw0, sw1)
        ibuf = (ibuf0, ibuf1)
        rows = (rows0, rows1)

        @pl.when(lax.axis_index("s") == 0)
        def _():
            pltpu.sync_copy(emb_hbm, tbl_s)

        plsc.subcore_barrier()

        def idx_load(sup, ss):
            start = pl.multiple_of(ibase + sup * _SUP_ROWS, _SUP_ROWS)
            pltpu.async_copy(
                idx_hbm.at[pl.ds(start, _SUP_ROWS)], ibuf[ss], si[ss])

        idx_load(0, 0)
        idx_load(1, 1)

        def sup_body(s2, carry):
            for ss in (0, 1):
                s = s2 * 2 + ss
                pltpu.make_async_copy(
                    idx_hbm.at[pl.ds(ibase, _SUP_ROWS)], ibuf[ss],
                    si[ss]).wait()

                def grp_body(q2, carry2):
                    for b in (0, 1):
                        q = q2 * 2 + b
                        g = s * _GPS + q
                        off = base + g * _GROUP
                        row_i = off // _SEQ
                        col_j = pl.multiple_of(off % _SEQ, _GROUP)
                        qs = q * _GPG

                        @pl.when(g >= 2)
                        def _():
                            pltpu.make_async_copy(
                                rows[b],
                                out_hbm.at[0, pl.ds(0, _GROUP)],
                                sw[b]).wait()

                        for j in range(_GPG):
                            pltpu.async_copy(
                                tbl_s.at[ibuf[ss].at[qs + j]],
                                rows[b].at[pl.ds(j * _ILIM, _ILIM)], sg[b])
                        for j in range(_GPG):
                            pltpu.make_async_copy(
                                tbl_s.at[ibuf[ss].at[qs + j]],
                                rows[b].at[pl.ds(j * _ILIM, _ILIM)],
                                sg[b]).wait()
                        pltpu.async_copy(
                            rows[b],
                            out_hbm.at[row_i, pl.ds(col_j, _GROUP)],
                            sw[b])
                    return carry2

                lax.fori_loop(0, _GPS // 2, grp_body, 0)

                idx_load(jnp.minimum(s + 2, _NSUP - 1), ss)
            return carry

        lax.fori_loop(0, _NSUP // 2, sup_body, 0)

        for b in (0, 1):
            pltpu.make_async_copy(
                idx_hbm.at[pl.ds(ibase, _SUP_ROWS)], ibuf[b], si[b]).wait()
            pltpu.make_async_copy(
                rows[b], out_hbm.at[0, pl.ds(0, _GROUP)], sw[b]).wait()

    out = _lookup(idx2, emb_p)
    return out[:, :, :_D]
